# baseline (speedup 1.0000x reference)
import functools

import jax
import jax.numpy as jnp
from jax import lax
from jax.experimental import pallas as pl
from jax.experimental.pallas import tpu as pltpu
from jax.experimental.pallas import tpu_sc as plsc

_BATCH = 4096
_VOCAB = 100000
_DIM = 64
_L = 16

_NC = 2
_NS = 16
_NW = _NC * _NS
_BPW = _BATCH // _NW

_BLK = 128
_MAIN_BLKS = _VOCAB // _BLK
_TAIL_LO = _MAIN_BLKS * _BLK
_TAIL_N = _VOCAB - _TAIL_LO
_BPT = 25
_MAX_START = _MAIN_BLKS - _BPT
_NSUB = 5
_SUBW = _BPT * _BLK // _NSUB
_SUBP = _SUBW + 8
_RING = 16
_NVEC = _BATCH // _L
_HCAP = _BATCH + _L

_mesh = plsc.VectorSubcoreMesh(core_axis_name="c", subcore_axis_name="s")


def _scalar(x):
    return x if getattr(x, "ndim", 0) == 0 else x[0]


def _lane_pick(vec, lane_mask):
    return jnp.sum(jnp.where(lane_mask, vec, 0))


@functools.partial(
    pl.kernel,
    mesh=_mesh,
    out_type=jax.ShapeDtypeStruct((_BATCH * _BLK,), jnp.float32),
    scratch_types=[
        pltpu.VMEM((_DIM, _SUBP), jnp.float32),
        pltpu.VMEM((_DIM, _SUBP), jnp.float32),
        pltpu.VMEM((_DIM, _BLK + 8), jnp.float32),
        pltpu.VMEM((_BATCH,), jnp.int32),
        pltpu.VMEM((_HCAP,), jnp.int32),
        pltpu.VMEM((_HCAP,), jnp.int32),
        pltpu.VMEM((_RING * _BLK,), jnp.float32),
        pltpu.SemaphoreType.DMA,
        pltpu.SemaphoreType.DMA,
        pltpu.SemaphoreType.DMA,
        pltpu.SemaphoreType.DMA,
    ],
    compiler_params=pltpu.CompilerParams(needs_layout_passes=False),
)
def _gather_k1(constt_hbm, idx_hbm, tailt_hbm, slots_hbm, sub_a, sub_b,
               tail_v, allidx_v, hit_rb, tl_rb, stage_v,
               sem_fa, sem_fb, sem_slot, sem_misc):
    wid = lax.axis_index("s") * _NC + lax.axis_index("c")
    start_blk = jnp.minimum(wid * _BPT, _MAX_START)
    start_lane = pl.multiple_of(start_blk * _BLK, _BLK)

    def fill(k):
        buf = sub_a if k % 2 == 0 else sub_b
        sem = sem_fa if k % 2 == 0 else sem_fb
        lo_k = pl.multiple_of(start_lane + k * _SUBW, _BLK)
        return pltpu.async_copy(
            constt_hbm.at[:, pl.ds(lo_k, _SUBW)],
            buf.at[:, pl.ds(0, _SUBW)], sem)

    fills = [fill(0), fill(1)]

    pltpu.sync_copy(idx_hbm, allidx_v)
    cp_t = pltpu.async_copy(
        tailt_hbm, tail_v.at[:, pl.ds(0, _BLK)], sem_misc)

    lanes = lax.iota(jnp.int32, _L)

    lo_all = start_lane
    hi_all = start_lane + _BPT * _BLK

    def scan_vec(v, carry):
        lc, tc = carry
        vec = allidx_v[pl.ds(v * _L, _L)]
        bvec = v * _L + lanes
        m = (vec >= lo_all) & (vec < hi_all)
        packed = vec * 4096 + bvec
        plsc.store_compressed(hit_rb.at[pl.ds(lc, _L)], packed, mask=m)
        lc = lc + _scalar(plsc.all_reduce_population_count(m))
        mt = vec >= _TAIL_LO
        tpacked = (vec - _TAIL_LO) * 4096 + bvec
        plsc.store_compressed(tl_rb.at[pl.ds(tc, _L)], tpacked, mask=mt)
        tc = tc + _scalar(plsc.all_reduce_population_count(mt))
        return lc, tc

    n_hits, n_tail = lax.fori_loop(
        0, _NVEC, scan_vec, (jnp.int32(0), jnp.int32(0)))

    def emit_column(buf, rloc, b, nd):
        @pl.when(nd >= _RING)
        def _():
            pltpu.make_async_copy(
                slots_hbm.at[pl.ds(0, _BLK)],
                stage_v.at[pl.ds(0, _BLK)], sem_slot).wait()
        slot = (nd & (_RING - 1)) * _BLK
        col = jnp.zeros((_L,), jnp.int32) + rloc
        for j in range(_DIM // _L):
            crow = j * _L + lanes
            stage_v[pl.ds(slot + j * _L, _L)] = plsc.load_gather(
                buf, [crow, col])
        pltpu.async_copy(
            stage_v.at[pl.ds(pl.multiple_of(slot, _BLK), _BLK)],
            slots_hbm.at[pl.ds(pl.multiple_of(b * _BLK, _BLK), _BLK)],
            sem_slot)
        return nd + 1

    def hit_loop(rvec, bvec, m0, nd, buf, lo):
        def cond(carry):
            m, _ = carry
            return _scalar(plsc.all_reduce_population_count(m)) > 0

        def body(carry):
            m, nd2 = carry
            l = _scalar(plsc.all_reduce_ffs(m))
            lm = lanes == l
            r = _lane_pick(rvec, lm)
            b = _lane_pick(bvec, lm)
            nd2 = emit_column(buf, r - lo, b, nd2)
            return m & ~lm, nd2

        _, nd = lax.while_loop(cond, body, (m0, nd))
        return nd

    n_dma = jnp.int32(0)
    hv_bound = lax.div(n_hits + _L - 1, jnp.int32(_L))

    for k in range(_NSUB):
        buf = sub_a if k % 2 == 0 else sub_b
        fills[k].wait()
        lo_k = start_lane + k * _SUBW

        def hvec_body(h, nd, _buf=buf, _lo=lo_k):
            pvec = hit_rb[pl.ds(h * _L, _L)]
            rvec = lax.shift_right_logical(pvec, 12)
            bvec = pvec & 4095
            valid = (h * _L + lanes) < n_hits
            m0 = valid & (rvec >= _lo) & (rvec < _lo + _SUBW)
            return hit_loop(rvec, bvec, m0, nd, _buf, _lo)

        n_dma = lax.fori_loop(0, hv_bound, hvec_body, n_dma)
        if k + 2 < _NSUB:
            fills.append(fill(k + 2))

    cp_t.wait()
    tv_bound = lax.div(n_tail + _L - 1, jnp.int32(_L))

    def tail_body(h, nd):
        pvec = tl_rb[pl.ds(h * _L, _L)]
        rvec = lax.shift_right_logical(pvec, 12)
        bvec = pvec & 4095
        m0 = (h * _L + lanes) < n_tail
        return hit_loop(rvec, bvec, m0, nd, tail_v, 0)

    n_dma = lax.fori_loop(0, tv_bound, tail_body, n_dma)

    def drain(i, carry):
        pltpu.make_async_copy(
            slots_hbm.at[pl.ds(0, _BLK)],
            stage_v.at[pl.ds(0, _BLK)], sem_slot).wait()
        return carry

    lax.fori_loop(0, jnp.minimum(n_dma, _RING), drain, 0)


@functools.partial(
    pl.kernel,
    mesh=_mesh,
    out_type=jax.ShapeDtypeStruct((_DIM, _BATCH), jnp.float32),
    scratch_types=[
        pltpu.VMEM((_BPW, _BLK + 8), jnp.float32),
        pltpu.VMEM((_DIM, _BPW + 8), jnp.float32),
        pltpu.VMEM((_DIM, _BPW + 8), jnp.float32),
        pltpu.SemaphoreType.DMA,
        pltpu.SemaphoreType.DMA,
    ],
    compiler_params=pltpu.CompilerParams(needs_layout_passes=False),
)
def _apply_k2(slots_hbm, xt_hbm, outt_hbm, slot_v, xt_v, out_v,
              sem_a, sem_b):
    wid = lax.axis_index("s") * _NC + lax.axis_index("c")
    base = wid * _BPW
    cp_s = pltpu.async_copy(
        slots_hbm.at[pl.ds(base, _BPW), :],
        slot_v.at[:, pl.ds(0, _BLK)], sem_a)
    cp_x = pltpu.async_copy(
        xt_hbm.at[:, pl.ds(base, _BPW)],
        xt_v.at[:, pl.ds(0, _BPW)], sem_b)
    cp_s.wait()
    cp_x.wait()

    lanes = lax.iota(jnp.int32, _L)
    rows_g = [g * _L + lanes for g in range(_BPW // _L)]

    def cbody(c, carry):
        cvec = jnp.zeros((_L,), jnp.int32) + c
        for g in range(_BPW // _L):
            gath = plsc.load_gather(slot_v, [rows_g[g], cvec])
            sl = pl.ds(g * _L, _L)
            out_v[c, sl] = gath + xt_v[c, sl]
        return carry

    lax.fori_loop(0, _DIM, cbody, 0)

    pltpu.sync_copy(out_v.at[:, pl.ds(0, _BPW)],
                    outt_hbm.at[:, pl.ds(base, _BPW)])


def kernel(x, const, indices):
    constt = const.T
    tailt = jnp.pad(constt[:, _TAIL_LO:], ((0, 0), (0, _BLK - _TAIL_N)))
    idx32 = indices.astype(jnp.int32)
    slots = _gather_k1(constt, idx32, tailt)
    outt = _apply_k2(slots.reshape(_BATCH, _BLK), x.T)
    return outt.T


# speedup vs baseline: 1.0049x; 1.0049x over previous
import functools

import jax
import jax.numpy as jnp
from jax import lax
from jax.experimental import pallas as pl
from jax.experimental.pallas import tpu as pltpu
from jax.experimental.pallas import tpu_sc as plsc

_BATCH = 4096
_VOCAB = 100000
_DIM = 64
_L = 16

_NC = 2
_NS = 16
_NW = _NC * _NS
_BPW = _BATCH // _NW

_BLK = 128
_MAIN_BLKS = _VOCAB // _BLK
_TAIL_LO = _MAIN_BLKS * _BLK
_TAIL_N = _VOCAB - _TAIL_LO
_BPT = 25
_MAX_START = _MAIN_BLKS - _BPT
_NSUB = 5
_SUBW = _BPT * _BLK // _NSUB
_SUBP = _SUBW + 8
_RING = 16
_NVEC = _BATCH // _L
_HCAP = _BATCH + _L

_mesh = plsc.VectorSubcoreMesh(core_axis_name="c", subcore_axis_name="s")


def _scalar(x):
    return x if getattr(x, "ndim", 0) == 0 else x[0]


def _lane_pick(vec, lane_mask):
    return jnp.sum(jnp.where(lane_mask, vec, 0))


@functools.partial(
    pl.kernel,
    mesh=_mesh,
    out_type=jax.ShapeDtypeStruct((_BATCH * _BLK,), jnp.float32),
    scratch_types=[
        pltpu.VMEM((_DIM, _SUBP), jnp.float32),
        pltpu.VMEM((_DIM, _SUBP), jnp.float32),
        pltpu.VMEM((_DIM, _BLK + 8), jnp.float32),
        pltpu.VMEM((_BATCH,), jnp.int32),
        pltpu.VMEM((_HCAP,), jnp.int32),
        pltpu.VMEM((_HCAP,), jnp.int32),
        pltpu.VMEM((_RING * _BLK,), jnp.float32),
        pltpu.SemaphoreType.DMA,
        pltpu.SemaphoreType.DMA,
        pltpu.SemaphoreType.DMA,
        pltpu.SemaphoreType.DMA,
    ],
    compiler_params=pltpu.CompilerParams(needs_layout_passes=False),
)
def _gather_k1(constt_hbm, idx_hbm, tailt_hbm, slots_hbm, sub_a, sub_b,
               tail_v, allidx_v, hit_rb, tl_rb, stage_v,
               sem_fa, sem_fb, sem_slot, sem_misc):
    wid = lax.axis_index("s") * _NC + lax.axis_index("c")
    start_blk = jnp.minimum(wid * _BPT, _MAX_START)
    start_lane = pl.multiple_of(start_blk * _BLK, _BLK)

    def fill(k):
        buf = sub_a if k % 2 == 0 else sub_b
        sem = sem_fa if k % 2 == 0 else sem_fb
        lo_k = pl.multiple_of(start_lane + k * _SUBW, _BLK)
        return pltpu.async_copy(
            constt_hbm.at[:, pl.ds(lo_k, _SUBW)],
            buf.at[:, pl.ds(0, _SUBW)], sem)

    fills = [fill(0), fill(1)]

    pltpu.sync_copy(idx_hbm, allidx_v)
    cp_t = pltpu.async_copy(
        tailt_hbm, tail_v.at[:, pl.ds(0, _BLK)], sem_misc)

    lanes = lax.iota(jnp.int32, _L)

    lo_all = start_lane
    hi_all = start_lane + _BPT * _BLK

    def scan_vec(v, carry):
        lc, tc = carry
        vec = allidx_v[pl.ds(v * _L, _L)]
        bvec = v * _L + lanes
        m = (vec >= lo_all) & (vec < hi_all)
        packed = vec * 4096 + bvec
        plsc.store_compressed(hit_rb.at[pl.ds(lc, _L)], packed, mask=m)
        lc = lc + _scalar(plsc.all_reduce_population_count(m))
        mt = vec >= _TAIL_LO
        tpacked = (vec - _TAIL_LO) * 4096 + bvec
        plsc.store_compressed(tl_rb.at[pl.ds(tc, _L)], tpacked, mask=mt)
        tc = tc + _scalar(plsc.all_reduce_population_count(mt))
        return lc, tc

    n_hits, n_tail = lax.fori_loop(
        0, _NVEC, scan_vec, (jnp.int32(0), jnp.int32(0)))

    def emit_column(buf, rloc, b, nd):
        @pl.when(nd >= _RING)
        def _():
            pltpu.make_async_copy(
                slots_hbm.at[pl.ds(0, _BLK)],
                stage_v.at[pl.ds(0, _BLK)], sem_slot).wait()
        slot = (nd & (_RING - 1)) * _BLK
        col = jnp.zeros((_L,), jnp.int32) + rloc
        for j in range(_DIM // _L):
            crow = j * _L + lanes
            stage_v[pl.ds(slot + j * _L, _L)] = plsc.load_gather(
                buf, [crow, col])
        pltpu.async_copy(
            stage_v.at[pl.ds(pl.multiple_of(slot, _BLK), _BLK)],
            slots_hbm.at[pl.ds(pl.multiple_of(b * _BLK, _BLK), _BLK)],
            sem_slot)
        return nd + 1

    def hit_loop(rvec, bvec, m0, nd, buf, lo):
        def cond(carry):
            m, _ = carry
            return _scalar(plsc.all_reduce_population_count(m)) > 0

        def body(carry):
            m, nd2 = carry
            l = _scalar(plsc.all_reduce_ffs(m))
            lm = lanes == l
            r = _lane_pick(rvec, lm)
            b = _lane_pick(bvec, lm)
            nd2 = emit_column(buf, r - lo, b, nd2)
            return m & ~lm, nd2

        _, nd = lax.while_loop(cond, body, (m0, nd))
        return nd

    n_dma = jnp.int32(0)
    hv_bound = lax.div(n_hits + _L - 1, jnp.int32(_L))

    for k in range(_NSUB):
        buf = sub_a if k % 2 == 0 else sub_b
        fills[k].wait()
        lo_k = start_lane + k * _SUBW

        def hvec_body(h, nd, _buf=buf, _lo=lo_k):
            pvec = hit_rb[pl.ds(h * _L, _L)]
            rvec = lax.shift_right_logical(pvec, 12)
            bvec = pvec & 4095
            valid = (h * _L + lanes) < n_hits
            m0 = valid & (rvec >= _lo) & (rvec < _lo + _SUBW)
            return hit_loop(rvec, bvec, m0, nd, _buf, _lo)

        n_dma = lax.fori_loop(0, hv_bound, hvec_body, n_dma)
        if k + 2 < _NSUB:
            fills.append(fill(k + 2))

    cp_t.wait()
    tv_bound = lax.div(n_tail + _L - 1, jnp.int32(_L))

    def tail_body(h, nd):
        pvec = tl_rb[pl.ds(h * _L, _L)]
        rvec = lax.shift_right_logical(pvec, 12)
        bvec = pvec & 4095
        m0 = (h * _L + lanes) < n_tail
        return hit_loop(rvec, bvec, m0, nd, tail_v, 0)

    n_dma = lax.fori_loop(0, tv_bound, tail_body, n_dma)

    def drain(i, carry):
        pltpu.make_async_copy(
            slots_hbm.at[pl.ds(0, _BLK)],
            stage_v.at[pl.ds(0, _BLK)], sem_slot).wait()
        return carry

    lax.fori_loop(0, jnp.minimum(n_dma, _RING), drain, 0)


@functools.partial(
    pl.kernel,
    mesh=_mesh,
    out_type=jax.ShapeDtypeStruct((_DIM, _BATCH), jnp.float32),
    scratch_types=[
        pltpu.VMEM((_BPW * _BLK,), jnp.float32),
        pltpu.VMEM((_DIM, _BPW + 8), jnp.float32),
        pltpu.VMEM((_DIM, _BPW + 8), jnp.float32),
        pltpu.SemaphoreType.DMA,
        pltpu.SemaphoreType.DMA,
    ],
    compiler_params=pltpu.CompilerParams(needs_layout_passes=False),
)
def _apply_k2(slots_hbm, xt_hbm, outt_hbm, slot_v, xt_v, out_v,
              sem_a, sem_b):
    wid = lax.axis_index("s") * _NC + lax.axis_index("c")
    base = wid * _BPW
    cp_s = pltpu.async_copy(
        slots_hbm.at[pl.ds(pl.multiple_of(base * _BLK, _BLK), _BPW * _BLK)],
        slot_v, sem_a)
    cp_x = pltpu.async_copy(
        xt_hbm.at[:, pl.ds(base, _BPW)],
        xt_v.at[:, pl.ds(0, _BPW)], sem_b)
    cp_s.wait()
    cp_x.wait()

    lanes = lax.iota(jnp.int32, _L)
    rows128_g = [(g * _L + lanes) * _BLK for g in range(_BPW // _L)]

    def cbody(c, carry):
        for g in range(_BPW // _L):
            gath = plsc.load_gather(slot_v, [rows128_g[g] + c])
            sl = pl.ds(g * _L, _L)
            out_v[c, sl] = gath + xt_v[c, sl]
        return carry

    lax.fori_loop(0, _DIM, cbody, 0)

    pltpu.sync_copy(out_v.at[:, pl.ds(0, _BPW)],
                    outt_hbm.at[:, pl.ds(base, _BPW)])


def kernel(x, const, indices):
    constt = const.T
    tailt = jnp.pad(constt[:, _TAIL_LO:], ((0, 0), (0, _BLK - _TAIL_N)))
    idx32 = indices.astype(jnp.int32)
    slots = _gather_k1(constt, idx32, tailt)
    outt = _apply_k2(slots, x.T)
    return outt.T
